# baseline (speedup 1.0000x reference)
import jax
import jax.numpy as jnp
from jax.experimental import pallas as pl
from jax.experimental.pallas import tpu as pltpu

SOS = 2
EOS = 3
MAX_LEN = 32
NEG = -1e30
V = 32000
E = 256
H = 512
L = 128
B = 16
T = MAX_LEN - 1
ROWS = 2 * B
VB = 6400
NV = V // VB
VB2 = 1280
NV2 = V // VB2
CH = 640
NCH = V // CH
CPB = VB // CH

_DOT = (((1,), (1,)), ((), ()))


def _dot(a, b):
    return jax.lax.dot_general(a, b, _DOT, preferred_element_type=jnp.float32)


def _enc_kernel(src_ref, emb_hbm, wih_ref, whh_ref, bih_ref, bhh_ref,
                ctx_ref, x_s, xp_s, h_s, sem):
    n_tok = L * B

    def issue(j, c):
        tok = src_ref[j]
        pltpu.make_async_copy(emb_hbm.at[tok], x_s.at[j], sem).start()
        return c

    jax.lax.fori_loop(0, n_tok, issue, 0)

    def waitf(j, c):
        tok = src_ref[j]
        pltpu.make_async_copy(emb_hbm.at[tok], x_s.at[j], sem).wait()
        return c

    jax.lax.fori_loop(0, n_tok, waitf, 0)

    xp_s[...] = _dot(x_s[...], wih_ref[...]) + bih_ref[...]
    h_s[...] = jnp.zeros((B, H), jnp.float32)

    def step(t, c):
        h = h_s[...]
        gi = xp_s[pl.ds(t * B, B), :]
        gh = _dot(h, whh_ref[...]) + bhh_ref[...]
        r = jax.nn.sigmoid(gi[:, 0:H] + gh[:, 0:H])
        z = jax.nn.sigmoid(gi[:, H:2 * H] + gh[:, H:2 * H])
        n = jnp.tanh(gi[:, 2 * H:3 * H] + r * gh[:, 2 * H:3 * H])
        h_s[...] = (1.0 - z) * n + z * h
        return c

    jax.lax.fori_loop(0, L, step, 0)
    ctx_ref[...] = h_s[...]


def _base_kernel(w_ref, ctx_ref, b_ref, base_ref, w16_ref):
    w = w_ref[...]
    w16_ref[...] = w[:, 0:E + H].astype(jnp.bfloat16)
    base_ref[...] = _dot(ctx_ref[...].astype(jnp.bfloat16),
                         w[:, E + H:E + 2 * H].astype(jnp.bfloat16)) + b_ref[0]


def _dec_kernel(slab_ref, base_ref, wih_ref, whh_ref, bih_ref, bhh_ref,
                ctx_ref, embdec_hbm, toks_ref, scores_ref,
                lg_s, h_s, nh_s, emb_s, tokv_s, toksm_s,
                prob_s, fin_s, th_s, sh_s, gsem, tsem):
    t = pl.program_id(0)
    v = pl.program_id(1)

    @pl.when(v == 0)
    def _head():
        @pl.when(t == 0)
        def _init():
            for rr in range(ROWS):
                toksm_s[rr, 0] = SOS
            cz = ctx_ref[...]
            h_s[...] = jnp.concatenate([cz, cz], 0)
            prob_s[...] = jnp.concatenate(
                [jnp.full((B, 1), 1.0, jnp.float32),
                 jnp.full((B, 1), NEG, jnp.float32)], 0)
            fin_s[...] = jnp.zeros((ROWS, 1), jnp.float32)
            col = jax.lax.broadcasted_iota(jnp.int32, (ROWS, MAX_LEN), 1)
            row = jax.lax.broadcasted_iota(jnp.int32, (ROWS, MAX_LEN), 0)
            th_s[...] = jnp.where(col == 0, SOS, EOS).astype(jnp.int32)
            sh_s[...] = jnp.where((col == 0) & (row < B), 1.0, 0.0)

        @pl.when(t > 0)
        def _wait_tok():
            pltpu.make_async_copy(tokv_s, toksm_s, tsem).wait()

        for rr in range(ROWS):
            tok = toksm_s[rr, 0]
            pltpu.make_async_copy(embdec_hbm.at[tok], emb_s.at[rr], gsem).start()
        for rr in range(ROWS):
            tok = toksm_s[rr, 0]
            pltpu.make_async_copy(embdec_hbm.at[tok], emb_s.at[rr], gsem).wait()

        cz2 = jnp.concatenate([ctx_ref[...], ctx_ref[...]], 0)
        x = jnp.concatenate([emb_s[...], cz2], 1)
        h = h_s[...]
        gi = _dot(x, wih_ref[...]) + bih_ref[...]
        gh = _dot(h, whh_ref[...]) + bhh_ref[...]
        r = jax.nn.sigmoid(gi[:, 0:H] + gh[:, 0:H])
        z = jax.nn.sigmoid(gi[:, H:2 * H] + gh[:, H:2 * H])
        n = jnp.tanh(gi[:, 2 * H:3 * H] + r * gh[:, 2 * H:3 * H])
        nh_s[...] = (1.0 - z) * n + z * h

    xh = jnp.concatenate([emb_s[...], nh_s[...]], 1).astype(jnp.bfloat16)
    bb = base_ref[v]
    blk = _dot(xh, slab_ref[...]) + jnp.concatenate([bb, bb], 0)
    for i in range(CPB):
        lg_s[v * CPB + i] = blk[:, i * CH:(i + 1) * CH]

    @pl.when(v == NV - 1)
    def _select():
        prob = prob_s[...]
        fin = fin_s[...]
        neg_inf = jnp.float32(-jnp.inf)
        bigi = jnp.int32(2 ** 30)

        def mx(vv, m):
            return jnp.maximum(m, jnp.max(lg_s[vv], axis=1, keepdims=True))

        m = jax.lax.fori_loop(0, NCH, mx,
                              jnp.full((ROWS, 1), neg_inf, jnp.float32))

        def sm(vv, s):
            e = jnp.exp(lg_s[vv] - m)
            lg_s[vv] = e
            return s + jnp.sum(e, axis=1, keepdims=True)

        s = jax.lax.fori_loop(0, NCH, sm, jnp.zeros((ROWS, 1), jnp.float32))

        def top2(vv, carry):
            bv1, bf1, bv2, bf2 = carry
            p = lg_s[vv] / s
            cand = p * prob
            cand = jnp.where(fin > 0, NEG, cand)
            lane = jax.lax.broadcasted_iota(jnp.int32, (ROWS, CH), 1) + vv * CH
            cand = jnp.where(lane == EOS,
                             jnp.where(fin > 0, prob, cand), cand)
            c1 = jnp.max(cand, axis=1, keepdims=True)
            i1 = jnp.min(jnp.where(cand == c1, lane, bigi), axis=1,
                         keepdims=True)
            msk = jnp.where(lane == i1, neg_inf, cand)
            c2 = jnp.max(msk, axis=1, keepdims=True)
            i2 = jnp.min(jnp.where(msk == c2, lane, bigi), axis=1,
                         keepdims=True)
            k1 = bv1 >= c1
            nv1 = jnp.where(k1, bv1, c1)
            nf1 = jnp.where(k1, bf1, i1)
            av = jnp.where(k1, bv2, bv1)
            af = jnp.where(k1, bf2, bf1)
            bv = jnp.where(k1, c1, c2)
            bf = jnp.where(k1, i1, i2)
            k2 = av >= bv
            return (nv1, nf1, jnp.where(k2, av, bv), jnp.where(k2, af, bf))

        init = (jnp.full((ROWS, 1), neg_inf, jnp.float32),
                jnp.zeros((ROWS, 1), jnp.int32),
                jnp.full((ROWS, 1), neg_inf, jnp.float32),
                jnp.zeros((ROWS, 1), jnp.int32))
        bv1, bf1, bv2, bf2 = jax.lax.fori_loop(0, NCH, top2, init)

        a1v, a1f = bv1[0:B], bf1[0:B]
        a2v, a2f = bv2[0:B], bf2[0:B]
        b1v, b1f = bv1[B:ROWS], bf1[B:ROWS] + V
        b2v, b2f = bv2[B:ROWS], bf2[B:ROWS] + V
        k1 = a1v >= b1v
        v1 = jnp.where(k1, a1v, b1v)
        f1 = jnp.where(k1, a1f, b1f)
        av = jnp.where(k1, a2v, a1v)
        af = jnp.where(k1, a2f, a1f)
        bv = jnp.where(k1, b1v, b2v)
        bf = jnp.where(k1, b1f, b2f)
        k2 = av >= bv
        v2 = jnp.where(k2, av, bv)
        f2 = jnp.where(k2, af, bf)

        par1 = (f1 >= V).astype(jnp.int32)
        tk1 = f1 - par1 * V
        par2 = (f2 >= V).astype(jnp.int32)
        tk2 = f2 - par2 * V

        fin0, fin1 = fin[0:B], fin[B:ROWS]
        th0, th1 = th_s[0:B, :], th_s[B:ROWS, :]
        sh0, sh1 = sh_s[0:B, :], sh_s[B:ROWS, :]
        nh0, nh1 = nh_s[0:B, :], nh_s[B:ROWS, :]
        col = jax.lax.broadcasted_iota(jnp.int32, (B, MAX_LEN), 1)
        tc = t + 1

        def pick(par, tk, val):
            selp = par == 0
            finp = jnp.where(selp, fin0, fin1)
            thp = jnp.where(selp, th0, th1)
            shp = jnp.where(selp, sh0, sh1)
            hp = jnp.where(selp, nh0, nh1)
            nth = jnp.where((col == tc) & (finp <= 0), tk, thp)
            nsh = jnp.where(col == tc, jnp.where(finp > 0, 0.0, val), shp)
            nfin = jnp.where(tk == EOS, 1.0, finp)
            return nth, nsh, hp, nfin

        nth1, nsh1, hp1, nfin1 = pick(par1, tk1, v1)
        nth2, nsh2, hp2, nfin2 = pick(par2, tk2, v2)

        h_s[0:B, :] = hp1
        h_s[B:ROWS, :] = hp2
        th_s[0:B, :] = nth1
        th_s[B:ROWS, :] = nth2
        sh_s[0:B, :] = nsh1
        sh_s[B:ROWS, :] = nsh2
        fin_s[...] = jnp.concatenate([nfin1, nfin2], 0)
        prob_s[...] = jnp.concatenate([v1, v2], 0)
        tokv_s[...] = jnp.concatenate([tk1, tk2], 0)

        @pl.when(t < T - 1)
        def _send_tok():
            pltpu.make_async_copy(tokv_s, toksm_s, tsem).start()

        @pl.when(t == T - 1)
        def _emit():
            best0 = v1 >= v2
            toks_ref[...] = jnp.where(best0, nth1, nth2)
            scores_ref[...] = jnp.where(best0, nsh1, nsh2)


def kernel(src, emb_enc, enc_Wih, enc_Whh, enc_bih, enc_bhh,
           emb_dec, dec_Wih, dec_Whh, dec_bih, dec_bhh, W_out, b_out):
    src32 = src.astype(jnp.int32).reshape(-1)

    ctx = pl.pallas_call(
        _enc_kernel,
        out_shape=jax.ShapeDtypeStruct((B, H), jnp.float32),
        in_specs=[
            pl.BlockSpec(memory_space=pltpu.SMEM),
            pl.BlockSpec(memory_space=pl.ANY),
            pl.BlockSpec(memory_space=pltpu.VMEM),
            pl.BlockSpec(memory_space=pltpu.VMEM),
            pl.BlockSpec(memory_space=pltpu.VMEM),
            pl.BlockSpec(memory_space=pltpu.VMEM),
        ],
        out_specs=pl.BlockSpec(memory_space=pltpu.VMEM),
        scratch_shapes=[
            pltpu.VMEM((L * B, E), jnp.float32),
            pltpu.VMEM((L * B, 3 * H), jnp.float32),
            pltpu.VMEM((B, H), jnp.float32),
            pltpu.SemaphoreType.DMA,
        ],
        name="enc_gru",
    )(src32, emb_enc, enc_Wih, enc_Whh,
      enc_bih.reshape(1, -1), enc_bhh.reshape(1, -1))

    base, w16 = pl.pallas_call(
        _base_kernel,
        out_shape=(jax.ShapeDtypeStruct((B, V), jnp.float32),
                   jax.ShapeDtypeStruct((V, E + H), jnp.bfloat16)),
        grid=(NV2,),
        in_specs=[
            pl.BlockSpec((VB2, E + 2 * H), lambda i: (i, 0)),
            pl.BlockSpec((B, H), lambda i: (0, 0)),
            pl.BlockSpec((1, 1, VB2), lambda i: (i, 0, 0)),
        ],
        out_specs=(pl.BlockSpec((B, VB2), lambda i: (0, i)),
                   pl.BlockSpec((VB2, E + H), lambda i: (i, 0))),
        compiler_params=pltpu.CompilerParams(
            dimension_semantics=("arbitrary",)),
        name="dec_base",
    )(W_out, ctx, b_out.reshape(NV2, 1, VB2))

    base3 = base.reshape(B, NV, VB).transpose(1, 0, 2)

    toks, scores = pl.pallas_call(
        _dec_kernel,
        out_shape=(jax.ShapeDtypeStruct((B, MAX_LEN), jnp.int32),
                   jax.ShapeDtypeStruct((B, MAX_LEN), jnp.float32)),
        grid=(T, NV),
        in_specs=[
            pl.BlockSpec((VB, E + H), lambda t, v: (v, 0)),
            pl.BlockSpec((NV, B, VB), lambda t, v: (0, 0, 0)),
            pl.BlockSpec((3 * H, E + H), lambda t, v: (0, 0)),
            pl.BlockSpec((3 * H, H), lambda t, v: (0, 0)),
            pl.BlockSpec((1, 3 * H), lambda t, v: (0, 0)),
            pl.BlockSpec((1, 3 * H), lambda t, v: (0, 0)),
            pl.BlockSpec((B, H), lambda t, v: (0, 0)),
            pl.BlockSpec(memory_space=pl.ANY),
        ],
        out_specs=(pl.BlockSpec((B, MAX_LEN), lambda t, v: (0, 0)),
                   pl.BlockSpec((B, MAX_LEN), lambda t, v: (0, 0))),
        scratch_shapes=[
            pltpu.VMEM((NCH, ROWS, CH), jnp.float32),
            pltpu.VMEM((ROWS, H), jnp.float32),
            pltpu.VMEM((ROWS, H), jnp.float32),
            pltpu.VMEM((ROWS, E), jnp.float32),
            pltpu.VMEM((ROWS, 1), jnp.int32),
            pltpu.SMEM((ROWS, 1), jnp.int32),
            pltpu.VMEM((ROWS, 1), jnp.float32),
            pltpu.VMEM((ROWS, 1), jnp.float32),
            pltpu.VMEM((ROWS, MAX_LEN), jnp.int32),
            pltpu.VMEM((ROWS, MAX_LEN), jnp.float32),
            pltpu.SemaphoreType.DMA,
            pltpu.SemaphoreType.DMA,
        ],
        compiler_params=pltpu.CompilerParams(
            dimension_semantics=("arbitrary", "arbitrary"),
            vmem_limit_bytes=56 * 1024 * 1024),
        name="beam_dec",
    )(w16, base3, dec_Wih, dec_Whh,
      dec_bih.reshape(1, -1), dec_bhh.reshape(1, -1), ctx, emb_dec)

    return toks, scores


# speedup vs baseline: 1.8895x; 1.8895x over previous
import jax
import jax.numpy as jnp
from jax.experimental import pallas as pl
from jax.experimental.pallas import tpu as pltpu

SOS = 2
EOS = 3
MAX_LEN = 32
NEG = -1e30
V = 32000
E = 256
H = 512
L = 128
B = 16
T = MAX_LEN - 1
ROWS = 2 * B
VB = 6400
NV = V // VB
VB2 = 1280
NV2 = V // VB2
CH = 640
NCH = V // CH
CPB = VB // CH

_DOT = (((1,), (1,)), ((), ()))


def _dot(a, b):
    return jax.lax.dot_general(a, b, _DOT, preferred_element_type=jnp.float32)


def _enc_kernel(src_ref, emb_hbm, wih_ref, whh_ref, bih_ref, bhh_ref,
                ctx_ref, x_s, xp_s, h_s, sem):
    n_tok = L * B

    def issue(j, c):
        tok = src_ref[j]
        pltpu.make_async_copy(emb_hbm.at[tok], x_s.at[j], sem).start()
        return c

    jax.lax.fori_loop(0, n_tok, issue, 0)

    def waitf(j, c):
        tok = src_ref[j]
        pltpu.make_async_copy(emb_hbm.at[tok], x_s.at[j], sem).wait()
        return c

    jax.lax.fori_loop(0, n_tok, waitf, 0)

    xp_s[...] = _dot(x_s[...], wih_ref[...]) + bih_ref[...]
    h_s[...] = jnp.zeros((B, H), jnp.float32)

    def step(t, c):
        h = h_s[...]
        gi = xp_s[pl.ds(t * B, B), :]
        gh = _dot(h, whh_ref[...]) + bhh_ref[...]
        r = jax.nn.sigmoid(gi[:, 0:H] + gh[:, 0:H])
        z = jax.nn.sigmoid(gi[:, H:2 * H] + gh[:, H:2 * H])
        n = jnp.tanh(gi[:, 2 * H:3 * H] + r * gh[:, 2 * H:3 * H])
        h_s[...] = (1.0 - z) * n + z * h
        return c

    jax.lax.fori_loop(0, L, step, 0)
    ctx_ref[...] = h_s[...]


def _base_kernel(w_ref, ctx_ref, b_ref, base_ref, w16_ref):
    w = w_ref[...]
    w16_ref[...] = w[:, 0:E + H].astype(jnp.bfloat16)
    base_ref[...] = _dot(ctx_ref[...].astype(jnp.bfloat16),
                         w[:, E + H:E + 2 * H].astype(jnp.bfloat16)) + b_ref[0]


def _dec_kernel(slab_ref, base_ref, wih_ref, whh_ref, bih_ref, bhh_ref,
                ctx_ref, embdec_hbm, toks_ref, scores_ref,
                lg_s, h_s, nh_s, emb_s, tokv_s, toksm_s,
                prob_s, fin_s, th_s, sh_s, gsem, tsem):
    t = pl.program_id(0)
    v = pl.program_id(1)

    @pl.when(v == 0)
    def _head():
        @pl.when(t == 0)
        def _init():
            for rr in range(ROWS):
                toksm_s[rr, 0] = SOS
            cz = ctx_ref[...]
            h_s[...] = jnp.concatenate([cz, cz], 0)
            prob_s[...] = jnp.concatenate(
                [jnp.full((B, 1), 1.0, jnp.float32),
                 jnp.full((B, 1), NEG, jnp.float32)], 0)
            fin_s[...] = jnp.zeros((ROWS, 1), jnp.float32)
            col = jax.lax.broadcasted_iota(jnp.int32, (ROWS, MAX_LEN), 1)
            row = jax.lax.broadcasted_iota(jnp.int32, (ROWS, MAX_LEN), 0)
            th_s[...] = jnp.where(col == 0, SOS, EOS).astype(jnp.int32)
            sh_s[...] = jnp.where((col == 0) & (row < B), 1.0, 0.0)

        @pl.when(t > 0)
        def _wait_tok():
            pltpu.make_async_copy(tokv_s, toksm_s, tsem).wait()

        for rr in range(ROWS):
            tok = toksm_s[rr, 0]
            pltpu.make_async_copy(embdec_hbm.at[tok], emb_s.at[rr], gsem).start()
        for rr in range(ROWS):
            tok = toksm_s[rr, 0]
            pltpu.make_async_copy(embdec_hbm.at[tok], emb_s.at[rr], gsem).wait()

        cz2 = jnp.concatenate([ctx_ref[...], ctx_ref[...]], 0)
        x = jnp.concatenate([emb_s[...], cz2], 1)
        h = h_s[...]
        gi = _dot(x, wih_ref[...]) + bih_ref[...]
        gh = _dot(h, whh_ref[...]) + bhh_ref[...]
        r = jax.nn.sigmoid(gi[:, 0:H] + gh[:, 0:H])
        z = jax.nn.sigmoid(gi[:, H:2 * H] + gh[:, H:2 * H])
        n = jnp.tanh(gi[:, 2 * H:3 * H] + r * gh[:, 2 * H:3 * H])
        nh_s[...] = (1.0 - z) * n + z * h

    xh = jnp.concatenate([emb_s[...], nh_s[...]], 1).astype(jnp.bfloat16)
    bb = base_ref[v]
    lg_s[v] = _dot(xh, slab_ref[...]) + jnp.concatenate([bb, bb], 0)

    @pl.when(v == NV - 1)
    def _select():
        prob = prob_s[...]
        fin = fin_s[...]
        neg_inf = jnp.float32(-jnp.inf)
        bigi = jnp.int32(2 ** 30)

        def mx(vv, m):
            return jnp.maximum(m, jnp.max(lg_s[vv], axis=1, keepdims=True))

        m = jax.lax.fori_loop(0, NV, mx,
                              jnp.full((ROWS, 1), neg_inf, jnp.float32))

        def sm(vv, s):
            e = jnp.exp(lg_s[vv] - m)
            lg_s[vv] = e
            return s + jnp.sum(e, axis=1, keepdims=True)

        s = jax.lax.fori_loop(0, NV, sm, jnp.zeros((ROWS, 1), jnp.float32))

        def top2(vv, carry):
            bv1, bf1, bv2, bf2 = carry
            p = lg_s[vv] / s
            cand = p * prob
            cand = jnp.where(fin > 0, NEG, cand)
            lane = jax.lax.broadcasted_iota(jnp.int32, (ROWS, VB), 1) + vv * VB
            cand = jnp.where(lane == EOS,
                             jnp.where(fin > 0, prob, cand), cand)
            c1 = jnp.max(cand, axis=1, keepdims=True)
            i1 = jnp.min(jnp.where(cand == c1, lane, bigi), axis=1,
                         keepdims=True)
            msk = jnp.where(lane == i1, neg_inf, cand)
            c2 = jnp.max(msk, axis=1, keepdims=True)
            i2 = jnp.min(jnp.where(msk == c2, lane, bigi), axis=1,
                         keepdims=True)
            k1 = bv1 >= c1
            nv1 = jnp.where(k1, bv1, c1)
            nf1 = jnp.where(k1, bf1, i1)
            av = jnp.where(k1, bv2, bv1)
            af = jnp.where(k1, bf2, bf1)
            bv = jnp.where(k1, c1, c2)
            bf = jnp.where(k1, i1, i2)
            k2 = av >= bv
            return (nv1, nf1, jnp.where(k2, av, bv), jnp.where(k2, af, bf))

        init = (jnp.full((ROWS, 1), neg_inf, jnp.float32),
                jnp.zeros((ROWS, 1), jnp.int32),
                jnp.full((ROWS, 1), neg_inf, jnp.float32),
                jnp.zeros((ROWS, 1), jnp.int32))
        bv1, bf1, bv2, bf2 = jax.lax.fori_loop(0, NV, top2, init)

        a1v, a1f = bv1[0:B], bf1[0:B]
        a2v, a2f = bv2[0:B], bf2[0:B]
        b1v, b1f = bv1[B:ROWS], bf1[B:ROWS] + V
        b2v, b2f = bv2[B:ROWS], bf2[B:ROWS] + V
        k1 = a1v >= b1v
        v1 = jnp.where(k1, a1v, b1v)
        f1 = jnp.where(k1, a1f, b1f)
        av = jnp.where(k1, a2v, a1v)
        af = jnp.where(k1, a2f, a1f)
        bv = jnp.where(k1, b1v, b2v)
        bf = jnp.where(k1, b1f, b2f)
        k2 = av >= bv
        v2 = jnp.where(k2, av, bv)
        f2 = jnp.where(k2, af, bf)

        par1 = (f1 >= V).astype(jnp.int32)
        tk1 = f1 - par1 * V
        par2 = (f2 >= V).astype(jnp.int32)
        tk2 = f2 - par2 * V

        fin0, fin1 = fin[0:B], fin[B:ROWS]
        th0, th1 = th_s[0:B, :], th_s[B:ROWS, :]
        sh0, sh1 = sh_s[0:B, :], sh_s[B:ROWS, :]
        nh0, nh1 = nh_s[0:B, :], nh_s[B:ROWS, :]
        col = jax.lax.broadcasted_iota(jnp.int32, (B, MAX_LEN), 1)
        tc = t + 1

        def pick(par, tk, val):
            selp = par == 0
            finp = jnp.where(selp, fin0, fin1)
            thp = jnp.where(selp, th0, th1)
            shp = jnp.where(selp, sh0, sh1)
            hp = jnp.where(selp, nh0, nh1)
            nth = jnp.where((col == tc) & (finp <= 0), tk, thp)
            nsh = jnp.where(col == tc, jnp.where(finp > 0, 0.0, val), shp)
            nfin = jnp.where(tk == EOS, 1.0, finp)
            return nth, nsh, hp, nfin

        nth1, nsh1, hp1, nfin1 = pick(par1, tk1, v1)
        nth2, nsh2, hp2, nfin2 = pick(par2, tk2, v2)

        h_s[0:B, :] = hp1
        h_s[B:ROWS, :] = hp2
        th_s[0:B, :] = nth1
        th_s[B:ROWS, :] = nth2
        sh_s[0:B, :] = nsh1
        sh_s[B:ROWS, :] = nsh2
        fin_s[...] = jnp.concatenate([nfin1, nfin2], 0)
        prob_s[...] = jnp.concatenate([v1, v2], 0)
        tokv_s[...] = jnp.concatenate([tk1, tk2], 0)

        @pl.when(t < T - 1)
        def _send_tok():
            pltpu.make_async_copy(tokv_s, toksm_s, tsem).start()

        @pl.when(t == T - 1)
        def _emit():
            best0 = v1 >= v2
            toks_ref[...] = jnp.where(best0, nth1, nth2)
            scores_ref[...] = jnp.where(best0, nsh1, nsh2)


def kernel(src, emb_enc, enc_Wih, enc_Whh, enc_bih, enc_bhh,
           emb_dec, dec_Wih, dec_Whh, dec_bih, dec_bhh, W_out, b_out):
    src32 = src.astype(jnp.int32).reshape(-1)

    ctx = pl.pallas_call(
        _enc_kernel,
        out_shape=jax.ShapeDtypeStruct((B, H), jnp.float32),
        in_specs=[
            pl.BlockSpec(memory_space=pltpu.SMEM),
            pl.BlockSpec(memory_space=pl.ANY),
            pl.BlockSpec(memory_space=pltpu.VMEM),
            pl.BlockSpec(memory_space=pltpu.VMEM),
            pl.BlockSpec(memory_space=pltpu.VMEM),
            pl.BlockSpec(memory_space=pltpu.VMEM),
        ],
        out_specs=pl.BlockSpec(memory_space=pltpu.VMEM),
        scratch_shapes=[
            pltpu.VMEM((L * B, E), jnp.float32),
            pltpu.VMEM((L * B, 3 * H), jnp.float32),
            pltpu.VMEM((B, H), jnp.float32),
            pltpu.SemaphoreType.DMA,
        ],
        name="enc_gru",
    )(src32, emb_enc, enc_Wih, enc_Whh,
      enc_bih.reshape(1, -1), enc_bhh.reshape(1, -1))

    base, w16 = pl.pallas_call(
        _base_kernel,
        out_shape=(jax.ShapeDtypeStruct((B, V), jnp.float32),
                   jax.ShapeDtypeStruct((V, E + H), jnp.bfloat16)),
        grid=(NV2,),
        in_specs=[
            pl.BlockSpec((VB2, E + 2 * H), lambda i: (i, 0)),
            pl.BlockSpec((B, H), lambda i: (0, 0)),
            pl.BlockSpec((1, 1, VB2), lambda i: (i, 0, 0)),
        ],
        out_specs=(pl.BlockSpec((B, VB2), lambda i: (0, i)),
                   pl.BlockSpec((VB2, E + H), lambda i: (i, 0))),
        compiler_params=pltpu.CompilerParams(
            dimension_semantics=("arbitrary",)),
        name="dec_base",
    )(W_out, ctx, b_out.reshape(NV2, 1, VB2))

    base3 = base.reshape(B, NV, VB).transpose(1, 0, 2)

    toks, scores = pl.pallas_call(
        _dec_kernel,
        out_shape=(jax.ShapeDtypeStruct((B, MAX_LEN), jnp.int32),
                   jax.ShapeDtypeStruct((B, MAX_LEN), jnp.float32)),
        grid=(T, NV),
        in_specs=[
            pl.BlockSpec((VB, E + H), lambda t, v: (v, 0)),
            pl.BlockSpec((NV, B, VB), lambda t, v: (0, 0, 0)),
            pl.BlockSpec((3 * H, E + H), lambda t, v: (0, 0)),
            pl.BlockSpec((3 * H, H), lambda t, v: (0, 0)),
            pl.BlockSpec((1, 3 * H), lambda t, v: (0, 0)),
            pl.BlockSpec((1, 3 * H), lambda t, v: (0, 0)),
            pl.BlockSpec((B, H), lambda t, v: (0, 0)),
            pl.BlockSpec(memory_space=pl.ANY),
        ],
        out_specs=(pl.BlockSpec((B, MAX_LEN), lambda t, v: (0, 0)),
                   pl.BlockSpec((B, MAX_LEN), lambda t, v: (0, 0))),
        scratch_shapes=[
            pltpu.VMEM((NV, ROWS, VB), jnp.float32),
            pltpu.VMEM((ROWS, H), jnp.float32),
            pltpu.VMEM((ROWS, H), jnp.float32),
            pltpu.VMEM((ROWS, E), jnp.float32),
            pltpu.VMEM((ROWS, 1), jnp.int32),
            pltpu.SMEM((ROWS, 1), jnp.int32),
            pltpu.VMEM((ROWS, 1), jnp.float32),
            pltpu.VMEM((ROWS, 1), jnp.float32),
            pltpu.VMEM((ROWS, MAX_LEN), jnp.int32),
            pltpu.VMEM((ROWS, MAX_LEN), jnp.float32),
            pltpu.SemaphoreType.DMA,
            pltpu.SemaphoreType.DMA,
        ],
        compiler_params=pltpu.CompilerParams(
            dimension_semantics=("arbitrary", "arbitrary"),
            vmem_limit_bytes=56 * 1024 * 1024),
        name="beam_dec",
    )(w16, base3, dec_Wih, dec_Whh,
      dec_bih.reshape(1, -1), dec_bhh.reshape(1, -1), ctx, emb_dec)

    return toks, scores


# speedup vs baseline: 1.9896x; 1.0530x over previous
import jax
import jax.numpy as jnp
from jax.experimental import pallas as pl
from jax.experimental.pallas import tpu as pltpu

SOS = 2
EOS = 3
MAX_LEN = 32
NEG = -1e30
V = 32000
E = 256
H = 512
L = 128
B = 16
T = MAX_LEN - 1
ROWS = 2 * B
VB = 6400
NV = V // VB
VB2 = 1280
NV2 = V // VB2
CH = 640
NCH = V // CH
CPB = VB // CH

_DOT = (((1,), (1,)), ((), ()))


def _dot(a, b):
    return jax.lax.dot_general(a, b, _DOT, preferred_element_type=jnp.float32)


def _enc_kernel(src_ref, emb_hbm, wih_ref, whh_ref, bih_ref, bhh_ref,
                ctx_ref, x_s, xp_s, h_s, sem):
    n_tok = L * B

    def issue(j, c):
        tok = src_ref[j]
        pltpu.make_async_copy(emb_hbm.at[tok], x_s.at[j], sem).start()
        return c

    jax.lax.fori_loop(0, n_tok, issue, 0)

    def waitf(j, c):
        tok = src_ref[j]
        pltpu.make_async_copy(emb_hbm.at[tok], x_s.at[j], sem).wait()
        return c

    jax.lax.fori_loop(0, n_tok, waitf, 0)

    xp_s[...] = _dot(x_s[...], wih_ref[...]) + bih_ref[...]
    h_s[...] = jnp.zeros((B, H), jnp.float32)

    def step(t, c):
        h = h_s[...]
        gi = xp_s[pl.ds(t * B, B), :]
        gh = _dot(h, whh_ref[...]) + bhh_ref[...]
        r = jax.nn.sigmoid(gi[:, 0:H] + gh[:, 0:H])
        z = jax.nn.sigmoid(gi[:, H:2 * H] + gh[:, H:2 * H])
        n = jnp.tanh(gi[:, 2 * H:3 * H] + r * gh[:, 2 * H:3 * H])
        h_s[...] = (1.0 - z) * n + z * h
        return c

    jax.lax.fori_loop(0, L, step, 0)
    ctx_ref[...] = h_s[...]


def _base_kernel(w_ref, ctx_ref, b_ref, base_ref, w16_ref):
    w = w_ref[...]
    w16_ref[...] = w[:, 0:E + H].astype(jnp.bfloat16)
    base_ref[...] = _dot(ctx_ref[...].astype(jnp.bfloat16),
                         w[:, E + H:E + 2 * H].astype(jnp.bfloat16)) + b_ref[0]


def _dec_kernel(slab_ref, base_ref, wih_ref, whh_ref, bih_ref, bhh_ref,
                ctx_ref, embdec_hbm, toks_ref, scores_ref,
                lg_s, h_s, nh_s, emb_s, tokv_s, toksm_s,
                prob_s, fin_s, th_s, sh_s, m_s, gsem, tsem):
    t = pl.program_id(0)
    v = pl.program_id(1)

    @pl.when(v == 0)
    def _head():
        @pl.when(t == 0)
        def _init():
            for rr in range(ROWS):
                toksm_s[rr, 0] = SOS
            cz = ctx_ref[...]
            h_s[...] = jnp.concatenate([cz, cz], 0)
            prob_s[...] = jnp.concatenate(
                [jnp.full((B, 1), 1.0, jnp.float32),
                 jnp.full((B, 1), NEG, jnp.float32)], 0)
            fin_s[...] = jnp.zeros((ROWS, 1), jnp.float32)
            col = jax.lax.broadcasted_iota(jnp.int32, (ROWS, MAX_LEN), 1)
            row = jax.lax.broadcasted_iota(jnp.int32, (ROWS, MAX_LEN), 0)
            th_s[...] = jnp.where(col == 0, SOS, EOS).astype(jnp.int32)
            sh_s[...] = jnp.where((col == 0) & (row < B), 1.0, 0.0)

        @pl.when(t == 0)
        def _first_gather():
            for rr in range(ROWS):
                tok = toksm_s[rr, 0]
                pltpu.make_async_copy(embdec_hbm.at[tok], emb_s.at[rr],
                                      gsem).start()

        for rr in range(ROWS):
            tok = toksm_s[rr, 0]
            pltpu.make_async_copy(embdec_hbm.at[tok], emb_s.at[rr], gsem).wait()

        cz2 = jnp.concatenate([ctx_ref[...], ctx_ref[...]], 0)
        x = jnp.concatenate([emb_s[...], cz2], 1)
        h = h_s[...]
        gi = _dot(x, wih_ref[...]) + bih_ref[...]
        gh = _dot(h, whh_ref[...]) + bhh_ref[...]
        r = jax.nn.sigmoid(gi[:, 0:H] + gh[:, 0:H])
        z = jax.nn.sigmoid(gi[:, H:2 * H] + gh[:, H:2 * H])
        n = jnp.tanh(gi[:, 2 * H:3 * H] + r * gh[:, 2 * H:3 * H])
        nh_s[...] = (1.0 - z) * n + z * h

    xh = jnp.concatenate([emb_s[...], nh_s[...]], 1).astype(jnp.bfloat16)
    bb = base_ref[v]
    blk = _dot(xh, slab_ref[...]) + jnp.concatenate([bb, bb], 0)
    lg_s[v] = blk
    bm = jnp.max(blk, axis=1, keepdims=True)
    m_s[...] = jnp.where(v == 0, bm, jnp.maximum(m_s[...], bm))

    @pl.when(v == NV - 1)
    def _select():
        prob = prob_s[...]
        fin = fin_s[...]
        neg_inf = jnp.float32(-jnp.inf)
        bigi = jnp.int32(2 ** 30)

        m = m_s[...]

        def sm(vv, s):
            e = jnp.exp(lg_s[vv] - m)
            lg_s[vv] = e
            return s + jnp.sum(e, axis=1, keepdims=True)

        s = jax.lax.fori_loop(0, NV, sm, jnp.zeros((ROWS, 1), jnp.float32))

        def top2(vv, carry):
            bv1, bf1, bv2, bf2 = carry
            p = lg_s[vv] / s
            cand = p * prob
            cand = jnp.where(fin > 0, NEG, cand)
            lane = jax.lax.broadcasted_iota(jnp.int32, (ROWS, VB), 1) + vv * VB
            cand = jnp.where(lane == EOS,
                             jnp.where(fin > 0, prob, cand), cand)
            c1 = jnp.max(cand, axis=1, keepdims=True)
            i1 = jnp.min(jnp.where(cand == c1, lane, bigi), axis=1,
                         keepdims=True)
            msk = jnp.where(lane == i1, neg_inf, cand)
            c2 = jnp.max(msk, axis=1, keepdims=True)
            i2 = jnp.min(jnp.where(msk == c2, lane, bigi), axis=1,
                         keepdims=True)
            k1 = bv1 >= c1
            nv1 = jnp.where(k1, bv1, c1)
            nf1 = jnp.where(k1, bf1, i1)
            av = jnp.where(k1, bv2, bv1)
            af = jnp.where(k1, bf2, bf1)
            bv = jnp.where(k1, c1, c2)
            bf = jnp.where(k1, i1, i2)
            k2 = av >= bv
            return (nv1, nf1, jnp.where(k2, av, bv), jnp.where(k2, af, bf))

        init = (jnp.full((ROWS, 1), neg_inf, jnp.float32),
                jnp.zeros((ROWS, 1), jnp.int32),
                jnp.full((ROWS, 1), neg_inf, jnp.float32),
                jnp.zeros((ROWS, 1), jnp.int32))
        bv1, bf1, bv2, bf2 = jax.lax.fori_loop(0, NV, top2, init)

        a1v, a1f = bv1[0:B], bf1[0:B]
        a2v, a2f = bv2[0:B], bf2[0:B]
        b1v, b1f = bv1[B:ROWS], bf1[B:ROWS] + V
        b2v, b2f = bv2[B:ROWS], bf2[B:ROWS] + V
        k1 = a1v >= b1v
        v1 = jnp.where(k1, a1v, b1v)
        f1 = jnp.where(k1, a1f, b1f)
        av = jnp.where(k1, a2v, a1v)
        af = jnp.where(k1, a2f, a1f)
        bv = jnp.where(k1, b1v, b2v)
        bf = jnp.where(k1, b1f, b2f)
        k2 = av >= bv
        v2 = jnp.where(k2, av, bv)
        f2 = jnp.where(k2, af, bf)

        par1 = (f1 >= V).astype(jnp.int32)
        tk1 = f1 - par1 * V
        par2 = (f2 >= V).astype(jnp.int32)
        tk2 = f2 - par2 * V

        fin0, fin1 = fin[0:B], fin[B:ROWS]
        th0, th1 = th_s[0:B, :], th_s[B:ROWS, :]
        sh0, sh1 = sh_s[0:B, :], sh_s[B:ROWS, :]
        nh0, nh1 = nh_s[0:B, :], nh_s[B:ROWS, :]
        col = jax.lax.broadcasted_iota(jnp.int32, (B, MAX_LEN), 1)
        tc = t + 1

        def pick(par, tk, val):
            selp = par == 0
            finp = jnp.where(selp, fin0, fin1)
            thp = jnp.where(selp, th0, th1)
            shp = jnp.where(selp, sh0, sh1)
            hp = jnp.where(selp, nh0, nh1)
            nth = jnp.where((col == tc) & (finp <= 0), tk, thp)
            nsh = jnp.where(col == tc, jnp.where(finp > 0, 0.0, val), shp)
            nfin = jnp.where(tk == EOS, 1.0, finp)
            return nth, nsh, hp, nfin

        nth1, nsh1, hp1, nfin1 = pick(par1, tk1, v1)
        nth2, nsh2, hp2, nfin2 = pick(par2, tk2, v2)

        h_s[0:B, :] = hp1
        h_s[B:ROWS, :] = hp2
        th_s[0:B, :] = nth1
        th_s[B:ROWS, :] = nth2
        sh_s[0:B, :] = nsh1
        sh_s[B:ROWS, :] = nsh2
        fin_s[...] = jnp.concatenate([nfin1, nfin2], 0)
        prob_s[...] = jnp.concatenate([v1, v2], 0)
        tokv_s[...] = jnp.concatenate([tk1, tk2], 0)

        @pl.when(t < T - 1)
        def _send_tok():
            cp = pltpu.make_async_copy(tokv_s, toksm_s, tsem)
            cp.start()
            cp.wait()
            for rr in range(ROWS):
                tok = toksm_s[rr, 0]
                pltpu.make_async_copy(embdec_hbm.at[tok], emb_s.at[rr],
                                      gsem).start()

        @pl.when(t == T - 1)
        def _emit():
            best0 = v1 >= v2
            toks_ref[...] = jnp.where(best0, nth1, nth2)
            scores_ref[...] = jnp.where(best0, nsh1, nsh2)


def kernel(src, emb_enc, enc_Wih, enc_Whh, enc_bih, enc_bhh,
           emb_dec, dec_Wih, dec_Whh, dec_bih, dec_bhh, W_out, b_out):
    src32 = src.astype(jnp.int32).reshape(-1)

    ctx = pl.pallas_call(
        _enc_kernel,
        out_shape=jax.ShapeDtypeStruct((B, H), jnp.float32),
        in_specs=[
            pl.BlockSpec(memory_space=pltpu.SMEM),
            pl.BlockSpec(memory_space=pl.ANY),
            pl.BlockSpec(memory_space=pltpu.VMEM),
            pl.BlockSpec(memory_space=pltpu.VMEM),
            pl.BlockSpec(memory_space=pltpu.VMEM),
            pl.BlockSpec(memory_space=pltpu.VMEM),
        ],
        out_specs=pl.BlockSpec(memory_space=pltpu.VMEM),
        scratch_shapes=[
            pltpu.VMEM((L * B, E), jnp.float32),
            pltpu.VMEM((L * B, 3 * H), jnp.float32),
            pltpu.VMEM((B, H), jnp.float32),
            pltpu.SemaphoreType.DMA,
        ],
        name="enc_gru",
    )(src32, emb_enc, enc_Wih, enc_Whh,
      enc_bih.reshape(1, -1), enc_bhh.reshape(1, -1))

    base, w16 = pl.pallas_call(
        _base_kernel,
        out_shape=(jax.ShapeDtypeStruct((B, V), jnp.float32),
                   jax.ShapeDtypeStruct((V, E + H), jnp.bfloat16)),
        grid=(NV2,),
        in_specs=[
            pl.BlockSpec((VB2, E + 2 * H), lambda i: (i, 0)),
            pl.BlockSpec((B, H), lambda i: (0, 0)),
            pl.BlockSpec((1, 1, VB2), lambda i: (i, 0, 0)),
        ],
        out_specs=(pl.BlockSpec((B, VB2), lambda i: (0, i)),
                   pl.BlockSpec((VB2, E + H), lambda i: (i, 0))),
        compiler_params=pltpu.CompilerParams(
            dimension_semantics=("arbitrary",)),
        name="dec_base",
    )(W_out, ctx, b_out.reshape(NV2, 1, VB2))

    base3 = base.reshape(B, NV, VB).transpose(1, 0, 2)

    toks, scores = pl.pallas_call(
        _dec_kernel,
        out_shape=(jax.ShapeDtypeStruct((B, MAX_LEN), jnp.int32),
                   jax.ShapeDtypeStruct((B, MAX_LEN), jnp.float32)),
        grid=(T, NV),
        in_specs=[
            pl.BlockSpec((VB, E + H), lambda t, v: (v, 0)),
            pl.BlockSpec((NV, B, VB), lambda t, v: (0, 0, 0)),
            pl.BlockSpec((3 * H, E + H), lambda t, v: (0, 0)),
            pl.BlockSpec((3 * H, H), lambda t, v: (0, 0)),
            pl.BlockSpec((1, 3 * H), lambda t, v: (0, 0)),
            pl.BlockSpec((1, 3 * H), lambda t, v: (0, 0)),
            pl.BlockSpec((B, H), lambda t, v: (0, 0)),
            pl.BlockSpec(memory_space=pl.ANY),
        ],
        out_specs=(pl.BlockSpec((B, MAX_LEN), lambda t, v: (0, 0)),
                   pl.BlockSpec((B, MAX_LEN), lambda t, v: (0, 0))),
        scratch_shapes=[
            pltpu.VMEM((NV, ROWS, VB), jnp.float32),
            pltpu.VMEM((ROWS, H), jnp.float32),
            pltpu.VMEM((ROWS, H), jnp.float32),
            pltpu.VMEM((ROWS, E), jnp.float32),
            pltpu.VMEM((ROWS, 1), jnp.int32),
            pltpu.SMEM((ROWS, 1), jnp.int32),
            pltpu.VMEM((ROWS, 1), jnp.float32),
            pltpu.VMEM((ROWS, 1), jnp.float32),
            pltpu.VMEM((ROWS, MAX_LEN), jnp.int32),
            pltpu.VMEM((ROWS, MAX_LEN), jnp.float32),
            pltpu.VMEM((ROWS, 1), jnp.float32),
            pltpu.SemaphoreType.DMA,
            pltpu.SemaphoreType.DMA,
        ],
        compiler_params=pltpu.CompilerParams(
            dimension_semantics=("arbitrary", "arbitrary"),
            vmem_limit_bytes=56 * 1024 * 1024),
        name="beam_dec",
    )(w16, base3, dec_Wih, dec_Whh,
      dec_bih.reshape(1, -1), dec_bhh.reshape(1, -1), ctx, emb_dec)

    return toks, scores
